# baseline (speedup 1.0000x reference)
import jax
import jax.numpy as jnp
from jax.experimental import pallas as pl

SEQ_LEN = 8192
OUT_DIM = 1024
BATCH = 4
BLOCK_SEQ = 512


def _add_kernel(in_ref, pos_ref, out_ref):
    out_ref[...] = in_ref[...] + pos_ref[...]


def kernel(inputs, pos_table):
    flat = inputs.reshape(BATCH * SEQ_LEN, OUT_DIM)
    n_seq = SEQ_LEN // BLOCK_SEQ
    out = pl.pallas_call(
        _add_kernel,
        grid=(n_seq, BATCH),
        in_specs=[
            pl.BlockSpec((BLOCK_SEQ, OUT_DIM), lambda i, b: (b * n_seq + i, 0)),
            pl.BlockSpec((BLOCK_SEQ, OUT_DIM), lambda i, b: (i, 0)),
        ],
        out_specs=pl.BlockSpec((BLOCK_SEQ, OUT_DIM), lambda i, b: (b * n_seq + i, 0)),
        out_shape=jax.ShapeDtypeStruct((BATCH * SEQ_LEN, OUT_DIM), inputs.dtype),
    )(flat, pos_table)
    return out.reshape(BATCH, SEQ_LEN, OUT_DIM)


# speedup vs baseline: 1.1496x; 1.1496x over previous
import jax
import jax.numpy as jnp
from jax.experimental import pallas as pl

SEQ_LEN = 8192
OUT_DIM = 1024
BATCH = 4
BLOCK_SEQ = 256


def _add_kernel(in_ref, pos_ref, out_ref):
    out_ref[...] = in_ref[...] + pos_ref[...][None, :, :]


def kernel(inputs, pos_table):
    grid = (SEQ_LEN // BLOCK_SEQ,)
    return pl.pallas_call(
        _add_kernel,
        grid=grid,
        in_specs=[
            pl.BlockSpec((BATCH, BLOCK_SEQ, OUT_DIM), lambda i: (0, i, 0)),
            pl.BlockSpec((BLOCK_SEQ, OUT_DIM), lambda i: (i, 0)),
        ],
        out_specs=pl.BlockSpec((BATCH, BLOCK_SEQ, OUT_DIM), lambda i: (0, i, 0)),
        out_shape=jax.ShapeDtypeStruct((BATCH, SEQ_LEN, OUT_DIM), inputs.dtype),
    )(inputs, pos_table)
